# baseline (speedup 1.0000x reference)
import jax
import jax.numpy as jnp
from jax import lax
from jax.experimental import pallas as pl
from jax.experimental.pallas import tpu as pltpu
from jax.experimental.pallas import tpu_sc as plsc

_N_BINS = 10
_N = 8388608
_M = 9.999999
_NC = 2
_NS = 16
_L = 16
_NW = _NC * _NS

_CHUNK = 16384
_NCHUNK_SC = 9
_PER_W = _NCHUNK_SC * _CHUNK
_SC_N = _NW * _PER_W
_VECS = _CHUNK // _L

_TC_COLS = 128
_TC_BR = 4096
_TC_GRID = (_N - _SC_N) // (_TC_BR * _TC_COLS)
_TC_B0 = _SC_N // (_TC_BR * _TC_COLS)
_NROWS = _N // _TC_COLS


def _sc_body(u_hbm, e_hbm, part_hbm, u0, e0, u1, e1, acc, sem0, sem1):
    wid = lax.axis_index("s") * _NC + lax.axis_index("c")
    base = wid * _PER_W
    lane = lax.iota(jnp.int32, _L)
    for i in range(_N_BINS):
        acc[i, :] = jnp.zeros((_L,), jnp.float32)

    bufs = ((u0, e0, sem0), (u1, e1, sem1))

    def start(g):
        ub, eb, sm = bufs[g & 1]
        off = base + g * _CHUNK
        cu = pltpu.async_copy(u_hbm.at[pl.ds(off, _CHUNK)], ub, sm)
        ce = pltpu.async_copy(e_hbm.at[pl.ds(off, _CHUNK)], eb, sm)
        return cu, ce

    pend = start(0)
    for g in range(_NCHUNK_SC):
        nxt = start(g + 1) if g + 1 < _NCHUNK_SC else None
        pend[0].wait()
        pend[1].wait()
        ub, eb, _ = bufs[g & 1]

        @plsc.parallel_loop(0, _VECS, unroll=8)
        def _body(i):
            u = ub[pl.ds(i * _L, _L)]
            e = eb[pl.ds(i * _L, _L)]
            row = (u * _M).astype(jnp.int32)
            plsc.addupdate_scatter(acc, [row, lane], u - e)

        pend = nxt
    pltpu.sync_copy(acc, part_hbm.at[wid])


_sc_pass = pl.kernel(
    _sc_body,
    out_type=jax.ShapeDtypeStruct((_NW, _N_BINS, _L), jnp.float32),
    mesh=plsc.VectorSubcoreMesh(core_axis_name="c", subcore_axis_name="s"),
    scratch_types=[
        pltpu.VMEM((_CHUNK,), jnp.float32),
        pltpu.VMEM((_CHUNK,), jnp.float32),
        pltpu.VMEM((_CHUNK,), jnp.float32),
        pltpu.VMEM((_CHUNK,), jnp.float32),
        pltpu.VMEM((_N_BINS, _L), jnp.float32),
        pltpu.SemaphoreType.DMA,
        pltpu.SemaphoreType.DMA,
    ],
    compiler_params=pltpu.CompilerParams(needs_layout_passes=False),
)


def _tc_body(u_ref, e_ref, out_ref):
    u = u_ref[...]
    e = e_ref[...]
    row = (u * _M).astype(jnp.int32)
    d = u - e
    s = jnp.stack([jnp.sum(jnp.where(row == b, d, 0.0), axis=0)
                   for b in range(_N_BINS)])
    out_ref[...] = s.reshape(1, _N_BINS, _TC_COLS)


_tc_pass = pl.pallas_call(
    _tc_body,
    grid=(_TC_GRID,),
    in_specs=[
        pl.BlockSpec((_TC_BR, _TC_COLS), lambda i: (_TC_B0 + i, 0)),
        pl.BlockSpec((_TC_BR, _TC_COLS), lambda i: (_TC_B0 + i, 0)),
    ],
    out_specs=pl.BlockSpec((1, _N_BINS, _TC_COLS), lambda i: (i, 0, 0)),
    out_shape=jax.ShapeDtypeStruct((_TC_GRID, _N_BINS, _TC_COLS), jnp.float32),
)


def _finish_body(part_sc_ref, part_tc_ref, out_ref):
    x = part_sc_ref[...]
    s_sc = jnp.sum(jnp.sum(x, axis=0), axis=1, keepdims=True)
    s_tc = jnp.sum(jnp.sum(part_tc_ref[...], axis=0), axis=1,
                   keepdims=True)
    out_ref[...] = (jnp.sum(jnp.abs(s_sc + s_tc)) * (1.0 / _N)).reshape(1, 1)


_finish = pl.pallas_call(
    _finish_body,
    out_shape=jax.ShapeDtypeStruct((1, 1), jnp.float32),
)


def kernel(uncertainties, errors):
    part_sc = _sc_pass(uncertainties, errors)
    u2 = uncertainties.reshape(_NROWS, _TC_COLS)
    e2 = errors.reshape(_NROWS, _TC_COLS)
    part_tc = _tc_pass(u2, e2)
    return _finish(part_sc, part_tc).reshape(1)


# speedup vs baseline: 1.0522x; 1.0522x over previous
import jax
import jax.numpy as jnp
from jax import lax
from jax.experimental import pallas as pl
from jax.experimental.pallas import tpu as pltpu
from jax.experimental.pallas import tpu_sc as plsc

_N_BINS = 10
_N = 8388608
_M = 9.999999
_NC = 2
_NS = 16
_L = 16
_NW = _NC * _NS

_CHUNK = 16384
_NCHUNK_SC = 8
_PER_W = _NCHUNK_SC * _CHUNK
_SC_N = _NW * _PER_W
_VECS = _CHUNK // _L

_TC_COLS = 128
_TC_BR = 4096
_TC_GRID = (_N - _SC_N) // (_TC_BR * _TC_COLS)
_TC_B0 = _SC_N // (_TC_BR * _TC_COLS)
_NROWS = _N // _TC_COLS


def _sc_body(u_hbm, e_hbm, part_hbm, u0, e0, u1, e1, acc, sem0, sem1):
    wid = lax.axis_index("s") * _NC + lax.axis_index("c")
    base = wid * _PER_W
    lane = lax.iota(jnp.int32, _L)
    for i in range(_N_BINS):
        acc[i, :] = jnp.zeros((_L,), jnp.float32)

    bufs = ((u0, e0, sem0), (u1, e1, sem1))

    def start(g):
        ub, eb, sm = bufs[g & 1]
        off = base + g * _CHUNK
        cu = pltpu.async_copy(u_hbm.at[pl.ds(off, _CHUNK)], ub, sm)
        ce = pltpu.async_copy(e_hbm.at[pl.ds(off, _CHUNK)], eb, sm)
        return cu, ce

    pend = start(0)
    for g in range(_NCHUNK_SC):
        nxt = start(g + 1) if g + 1 < _NCHUNK_SC else None
        pend[0].wait()
        pend[1].wait()
        ub, eb, _ = bufs[g & 1]

        @plsc.parallel_loop(0, _VECS, unroll=8)
        def _body(i):
            u = ub[pl.ds(i * _L, _L)]
            e = eb[pl.ds(i * _L, _L)]
            row = (u * _M).astype(jnp.int32)
            plsc.addupdate_scatter(acc, [row, lane], u - e)

        pend = nxt
    pltpu.sync_copy(acc, part_hbm.at[wid])


_sc_pass = pl.kernel(
    _sc_body,
    out_type=jax.ShapeDtypeStruct((_NW, _N_BINS, _L), jnp.float32),
    mesh=plsc.VectorSubcoreMesh(core_axis_name="c", subcore_axis_name="s"),
    scratch_types=[
        pltpu.VMEM((_CHUNK,), jnp.float32),
        pltpu.VMEM((_CHUNK,), jnp.float32),
        pltpu.VMEM((_CHUNK,), jnp.float32),
        pltpu.VMEM((_CHUNK,), jnp.float32),
        pltpu.VMEM((_N_BINS, _L), jnp.float32),
        pltpu.SemaphoreType.DMA,
        pltpu.SemaphoreType.DMA,
    ],
    compiler_params=pltpu.CompilerParams(needs_layout_passes=False),
)


def _tc_body(u_ref, e_ref, out_ref):
    u = u_ref[...]
    e = e_ref[...]
    row = (u * _M).astype(jnp.int32)
    d = u - e
    s = jnp.stack([jnp.sum(jnp.where(row == b, d, 0.0), axis=0)
                   for b in range(_N_BINS)])
    out_ref[...] = s.reshape(1, _N_BINS, _TC_COLS)


_tc_pass = pl.pallas_call(
    _tc_body,
    grid=(_TC_GRID,),
    in_specs=[
        pl.BlockSpec((_TC_BR, _TC_COLS), lambda i: (_TC_B0 + i, 0)),
        pl.BlockSpec((_TC_BR, _TC_COLS), lambda i: (_TC_B0 + i, 0)),
    ],
    out_specs=pl.BlockSpec((1, _N_BINS, _TC_COLS), lambda i: (i, 0, 0)),
    out_shape=jax.ShapeDtypeStruct((_TC_GRID, _N_BINS, _TC_COLS), jnp.float32),
)


def _finish_body(part_sc_ref, part_tc_ref, out_ref):
    x = part_sc_ref[...]
    s_sc = jnp.sum(jnp.sum(x, axis=0), axis=1, keepdims=True)
    s_tc = jnp.sum(jnp.sum(part_tc_ref[...], axis=0), axis=1,
                   keepdims=True)
    out_ref[...] = (jnp.sum(jnp.abs(s_sc + s_tc)) * (1.0 / _N)).reshape(1, 1)


_finish = pl.pallas_call(
    _finish_body,
    out_shape=jax.ShapeDtypeStruct((1, 1), jnp.float32),
)


def kernel(uncertainties, errors):
    part_sc = _sc_pass(uncertainties, errors)
    u2 = uncertainties.reshape(_NROWS, _TC_COLS)
    e2 = errors.reshape(_NROWS, _TC_COLS)
    part_tc = _tc_pass(u2, e2)
    return _finish(part_sc, part_tc).reshape(1)
